# baseline (speedup 1.0000x reference)
import functools

import jax
import jax.numpy as jnp
from jax import lax
from jax.experimental import pallas as pl
from jax.experimental.pallas import tpu as pltpu
from jax.experimental.pallas import tpu_sc as plsc

INPUT_DIM = 768
OUTPUT_DIM = 768
NUM_EXPERTS = 16
TOP_K = 2
BATCH = 64

L = 16
PK = BATCH * TOP_K
NW = 32
R_PER = OUTPUT_DIM // NW

BI = 16


def _iota():
    return lax.broadcasted_iota(jnp.int32, (L,), 0)


def _splat(x):
    return jnp.full((L,), x, jnp.int32)


def _sc_body(bp_hbm, bi_hbm, bb_hbm, outbT_hbm,
             bp_v, bi_v, bb_v, outb_v):
    cid = lax.axis_index("c")
    sid = lax.axis_index("s")
    wid = sid * 2 + cid
    iov = _iota()
    r0 = wid * R_PER

    pltpu.sync_copy(bp_hbm.at[pl.ds(r0 * PK, R_PER * PK)], bp_v)
    pltpu.sync_copy(bi_hbm.at[pl.ds(r0 * PK, R_PER * PK)], bi_v)
    pltpu.sync_copy(bb_hbm.at[pl.ds(r0 * NUM_EXPERTS, R_PER * NUM_EXPERTS)],
                    bb_v)

    for rl in range(R_PER):
        ebase = _splat(rl * NUM_EXPERTS)
        for g in range(BATCH // L):
            src = _splat(rl * PK + g * 2 * L) + iov * 2
            p0 = plsc.load_gather(bp_v, [src])
            e0 = plsc.load_gather(bi_v, [src])
            p1 = plsc.load_gather(bp_v, [src + 1])
            e1 = plsc.load_gather(bi_v, [src + 1])
            v0 = plsc.load_gather(bb_v, [ebase + e0])
            v1 = plsc.load_gather(bb_v, [ebase + e1])
            outb_v[rl, pl.ds(g * L, L)] = p0 * v0 + p1 * v1

    pltpu.sync_copy(outb_v, outbT_hbm.at[pl.ds(r0, R_PER)])


def _weight_body(wp_ref, wi_ref, bank_ref, out_ref):
    e_iota = lax.broadcasted_iota(jnp.int32, (BI, BATCH, NUM_EXPERTS), 2)
    s = jnp.where(wi_ref[:, :, 0][:, :, None] == e_iota,
                  wp_ref[:, :, 0][:, :, None], 0.0)
    s = s + jnp.where(wi_ref[:, :, 1][:, :, None] == e_iota,
                      wp_ref[:, :, 1][:, :, None], 0.0)
    res = lax.dot_general(
        s, bank_ref[...],
        dimension_numbers=(((2,), (1,)), ((0,), (0,))),
        preferred_element_type=jnp.float32)
    out_ref[...] = jnp.transpose(res, (1, 0, 2))


@jax.jit
def kernel(weight_probs, weight_indices, bias_probs, bias_indices,
           weight_bank, bias_bank):
    bp = bias_probs.reshape(-1)
    bi = bias_indices.reshape(-1)
    bb = bias_bank.reshape(-1)

    mesh = plsc.VectorSubcoreMesh(core_axis_name="c", subcore_axis_name="s")
    outbT = pl.kernel(
        _sc_body,
        out_type=jax.ShapeDtypeStruct((OUTPUT_DIM, BATCH), jnp.float32),
        mesh=mesh,
        compiler_params=pltpu.CompilerParams(needs_layout_passes=False),
        scratch_types=(
            pltpu.VMEM((R_PER * PK,), jnp.float32),
            pltpu.VMEM((R_PER * PK,), jnp.int32),
            pltpu.VMEM((R_PER * NUM_EXPERTS,), jnp.float32),
            pltpu.VMEM((R_PER, BATCH), jnp.float32),
        ),
    )(bp, bi, bb)

    nblk = INPUT_DIM // BI
    dxb = pl.BlockSpec((BI, BATCH, TOP_K), lambda i: (i, 0, 0))
    weight_mixture = pl.pallas_call(
        _weight_body,
        grid=(nblk,),
        in_specs=[
            dxb, dxb,
            pl.BlockSpec((BI, NUM_EXPERTS, OUTPUT_DIM), lambda i: (i, 0, 0)),
        ],
        out_specs=pl.BlockSpec((BATCH, BI, OUTPUT_DIM), lambda i: (0, i, 0)),
        out_shape=jax.ShapeDtypeStruct((BATCH, INPUT_DIM, OUTPUT_DIM),
                                       jnp.float32),
    )(weight_probs, weight_indices, weight_bank)

    return weight_mixture, outbT.T


# speedup vs baseline: 1.2602x; 1.2602x over previous
import functools

import jax
import jax.numpy as jnp
from jax import lax
from jax.experimental import pallas as pl
from jax.experimental.pallas import tpu as pltpu
from jax.experimental.pallas import tpu_sc as plsc

INPUT_DIM = 768
OUTPUT_DIM = 768
NUM_EXPERTS = 16
TOP_K = 2
BATCH = 64

L = 16
PK = BATCH * TOP_K
NW = 32
R_PER = OUTPUT_DIM // NW

BI = 16


def _iota():
    return lax.broadcasted_iota(jnp.int32, (L,), 0)


def _splat(x):
    return jnp.full((L,), x, jnp.int32)


def _sc_body(bp_hbm, bi_hbm, bb_hbm, outbT_hbm,
             bp_v, bi_v, bb_v, outb_v):
    cid = lax.axis_index("c")
    sid = lax.axis_index("s")
    wid = sid * 2 + cid
    iov = _iota()
    r0 = wid * R_PER

    pltpu.sync_copy(bp_hbm.at[pl.ds(r0 * PK, R_PER * PK)], bp_v)
    pltpu.sync_copy(bi_hbm.at[pl.ds(r0 * PK, R_PER * PK)], bi_v)
    pltpu.sync_copy(bb_hbm.at[pl.ds(r0 * NUM_EXPERTS, R_PER * NUM_EXPERTS)],
                    bb_v)

    for rl in range(R_PER):
        ebase = _splat(rl * NUM_EXPERTS)
        for g in range(BATCH // L):
            src = _splat(rl * PK + g * 2 * L) + iov * 2
            p0 = plsc.load_gather(bp_v, [src])
            e0 = plsc.load_gather(bi_v, [src])
            p1 = plsc.load_gather(bp_v, [src + 1])
            e1 = plsc.load_gather(bi_v, [src + 1])
            v0 = plsc.load_gather(bb_v, [ebase + e0])
            v1 = plsc.load_gather(bb_v, [ebase + e1])
            outb_v[rl, pl.ds(g * L, L)] = p0 * v0 + p1 * v1

    pltpu.sync_copy(outb_v, outbT_hbm.at[pl.ds(r0, R_PER)])


def _weight_body(wp0_ref, wp1_ref, wi0_ref, wi1_ref, bank_ref, out_ref):
    e_iota = lax.broadcasted_iota(jnp.int32, (BI, BATCH, NUM_EXPERTS), 2)
    s = jnp.where(wi0_ref[...][:, :, None] == e_iota,
                  wp0_ref[...][:, :, None], 0.0)
    s = s + jnp.where(wi1_ref[...][:, :, None] == e_iota,
                      wp1_ref[...][:, :, None], 0.0)
    res = lax.dot_general(
        s, bank_ref[...],
        dimension_numbers=(((2,), (1,)), ((0,), (0,))),
        preferred_element_type=jnp.float32)
    out_ref[...] = jnp.transpose(res, (1, 0, 2))


@jax.jit
def kernel(weight_probs, weight_indices, bias_probs, bias_indices,
           weight_bank, bias_bank):
    bp = bias_probs.reshape(-1)
    bi = bias_indices.reshape(-1)
    bb = bias_bank.reshape(-1)

    mesh = plsc.VectorSubcoreMesh(core_axis_name="c", subcore_axis_name="s")
    outbT = pl.kernel(
        _sc_body,
        out_type=jax.ShapeDtypeStruct((OUTPUT_DIM, BATCH), jnp.float32),
        mesh=mesh,
        compiler_params=pltpu.CompilerParams(needs_layout_passes=False),
        scratch_types=(
            pltpu.VMEM((R_PER * PK,), jnp.float32),
            pltpu.VMEM((R_PER * PK,), jnp.int32),
            pltpu.VMEM((R_PER * NUM_EXPERTS,), jnp.float32),
            pltpu.VMEM((R_PER, BATCH), jnp.float32),
        ),
    )(bp, bi, bb)

    wp0, wp1 = weight_probs[:, :, 0], weight_probs[:, :, 1]
    wi0, wi1 = weight_indices[:, :, 0], weight_indices[:, :, 1]
    nblk = INPUT_DIM // BI
    dxb = pl.BlockSpec((BI, BATCH), lambda i: (i, 0))
    weight_mixture = pl.pallas_call(
        _weight_body,
        grid=(nblk,),
        in_specs=[
            dxb, dxb, dxb, dxb,
            pl.BlockSpec((BI, NUM_EXPERTS, OUTPUT_DIM), lambda i: (i, 0, 0)),
        ],
        out_specs=pl.BlockSpec((BATCH, BI, OUTPUT_DIM), lambda i: (0, i, 0)),
        out_shape=jax.ShapeDtypeStruct((BATCH, INPUT_DIM, OUTPUT_DIM),
                                       jnp.float32),
    )(wp0, wp1, wi0, wi1, weight_bank)

    return weight_mixture, outbT.T
